# baseline (speedup 1.0000x reference)
import functools

import jax
import jax.numpy as jnp
from jax import lax
from jax.experimental import pallas as pl
from jax.experimental.pallas import tpu as pltpu
from jax.experimental.pallas import tpu_sc as plsc

_D, _H, _W = 128, 128, 96
_N = 1_000_000
_NW = 32
_P = 32768
_C = 2048
_NCHUNK = _P // _C
_MAGIC = 8388608.0


def _to_index(v, a, scale):
    xn = v / a
    xn = (xn + 1.0) / 2.0
    t = xn * 2.0 - 1.0
    w = (t + 1.0) * 0.5 * scale
    w = jnp.maximum(w, 0.0)
    w = jnp.minimum(w, scale)
    r = (w + _MAGIC) - _MAGIC
    return r.astype(jnp.int32)


def _body(x_hbm, vol_hbm, aabb_hbm, out_hbm, xbuf, abuf, idxbuf, outbuf, sem):
    cid = lax.axis_index("c")
    sid = lax.axis_index("s")
    wid = sid * 2 + cid
    base = jnp.minimum(wid * _P, _N - _P)

    pltpu.sync_copy(aabb_hbm, abuf)
    a0 = abuf[0, :]
    a1 = abuf[1, :]
    a2 = abuf[2, :]
    iota3 = lax.iota(jnp.int32, 16) * 3

    def chunk_body(j, carry):
        cbase = base + j * _C
        pltpu.sync_copy(x_hbm.at[pl.ds(cbase * 3, 3 * _C)], xbuf)

        def pt_body(i, c2):
            bi = iota3 + i * 48
            vx = plsc.load_gather(xbuf, [bi])
            vy = plsc.load_gather(xbuf, [bi + 1])
            vz = plsc.load_gather(xbuf, [bi + 2])
            iz = _to_index(vx, a0, 127.0)
            iy = _to_index(vy, a1, 127.0)
            ix = _to_index(vz, a2, 95.0)
            flat = (iz * _H + iy) * _W + ix
            row = i // 8
            col = (i % 8) * 16
            idxbuf[row, pl.ds(col, 16)] = flat
            return c2

        lax.fori_loop(0, _C // 16, pt_body, 0)

        copies = [
            pltpu.async_copy(vol_hbm.at[idxbuf.at[r]],
                             outbuf.at[pl.ds(r * 128, 128)], sem)
            for r in range(16)
        ]
        for cpy in copies:
            cpy.wait()
        pltpu.sync_copy(outbuf, out_hbm.at[pl.ds(cbase, _C)])
        return carry

    lax.fori_loop(0, _NCHUNK, chunk_body, 0)


@jax.jit
def _sc_lookup(xf, vol, ab):
    mesh = plsc.VectorSubcoreMesh(core_axis_name="c", subcore_axis_name="s")
    run = functools.partial(
        pl.kernel,
        mesh=mesh,
        compiler_params=pltpu.CompilerParams(needs_layout_passes=False),
        out_type=jax.ShapeDtypeStruct((_N,), jnp.float32),
        scratch_types=[
            pltpu.VMEM((3 * _C,), jnp.float32),
            pltpu.VMEM((3, 16), jnp.float32),
            pltpu.VMEM((16, 128), jnp.int32),
            pltpu.VMEM((_C,), jnp.float32),
            pltpu.SemaphoreType.DMA,
        ],
    )(_body)
    return run(xf, vol, ab)


def kernel(x, aabb, image, voxels):
    vol = image.reshape(-1)
    xf = x.reshape(-1)
    ab = jnp.broadcast_to(aabb[:, None], (3, 16))
    out = _sc_lookup(xf, vol, ab)
    return out[:, None]


# speedup vs baseline: 12.1811x; 12.1811x over previous
import functools

import jax
import jax.numpy as jnp
from jax import lax
from jax.experimental import pallas as pl
from jax.experimental.pallas import tpu as pltpu
from jax.experimental.pallas import tpu_sc as plsc

_D, _H, _W = 128, 128, 96
_N = 1_000_000
_NW = 32
_P = 32768
_C = 2048
_NCHUNK = _P // _C
_MAGIC = 8388608.0


def _to_index(v, a, scale):
    xn = v / a
    xn = (xn + 1.0) / 2.0
    t = xn * 2.0 - 1.0
    w = (t + 1.0) * 0.5 * scale
    w = jnp.maximum(w, 0.0)
    w = jnp.minimum(w, scale)
    r = (w + _MAGIC) - _MAGIC
    return r.astype(jnp.int32)


def _body(x0_hbm, x1_hbm, x2_hbm, vol_hbm, aabb_hbm, out_hbm,
          xb0, xb1, xb2, abuf, idxbuf, outbuf, sem):
    cid = lax.axis_index("c")
    sid = lax.axis_index("s")
    wid = sid * 2 + cid
    base = jnp.minimum(wid * _P, _N - _P)

    pltpu.sync_copy(aabb_hbm, abuf)
    a0 = abuf[0, :]
    a1 = abuf[1, :]
    a2 = abuf[2, :]

    def chunk_body(j, carry):
        cbase = base + j * _C
        pltpu.sync_copy(x0_hbm.at[pl.ds(cbase, _C)], xb0)
        pltpu.sync_copy(x1_hbm.at[pl.ds(cbase, _C)], xb1)
        pltpu.sync_copy(x2_hbm.at[pl.ds(cbase, _C)], xb2)

        def pt_body(i, c2):
            off = i * 16
            vx = xb0[pl.ds(off, 16)]
            vy = xb1[pl.ds(off, 16)]
            vz = xb2[pl.ds(off, 16)]
            iz = _to_index(vx, a0, 127.0)
            iy = _to_index(vy, a1, 127.0)
            ix = _to_index(vz, a2, 95.0)
            flat = (iz * _H + iy) * _W + ix
            row = i // 8
            col = (i % 8) * 16
            idxbuf[row, pl.ds(col, 16)] = flat
            return c2

        lax.fori_loop(0, _C // 16, pt_body, 0)

        copies = [
            pltpu.async_copy(vol_hbm.at[idxbuf.at[r]],
                             outbuf.at[pl.ds(r * 128, 128)], sem)
            for r in range(16)
        ]
        for cpy in copies:
            cpy.wait()
        pltpu.sync_copy(outbuf, out_hbm.at[pl.ds(cbase, _C)])
        return carry

    lax.fori_loop(0, _NCHUNK, chunk_body, 0)


@jax.jit
def _sc_lookup(x0, x1, x2, vol, ab):
    mesh = plsc.VectorSubcoreMesh(core_axis_name="c", subcore_axis_name="s")
    run = functools.partial(
        pl.kernel,
        mesh=mesh,
        compiler_params=pltpu.CompilerParams(needs_layout_passes=False),
        out_type=jax.ShapeDtypeStruct((_N,), jnp.float32),
        scratch_types=[
            pltpu.VMEM((_C,), jnp.float32),
            pltpu.VMEM((_C,), jnp.float32),
            pltpu.VMEM((_C,), jnp.float32),
            pltpu.VMEM((3, 16), jnp.float32),
            pltpu.VMEM((16, 128), jnp.int32),
            pltpu.VMEM((_C,), jnp.float32),
            pltpu.SemaphoreType.DMA,
        ],
    )(_body)
    return run(x0, x1, x2, vol, ab)


def kernel(x, aabb, image, voxels):
    vol = image.reshape(-1)
    ab = jnp.broadcast_to(aabb[:, None], (3, 16))
    out = _sc_lookup(x[:, 0], x[:, 1], x[:, 2], vol, ab)
    return out[:, None]


# speedup vs baseline: 13.7176x; 1.1261x over previous
import functools

import jax
import jax.numpy as jnp
from jax import lax
from jax.experimental import pallas as pl
from jax.experimental.pallas import tpu as pltpu
from jax.experimental.pallas import tpu_sc as plsc

_D, _H, _W = 128, 128, 96
_N = 1_000_000
_NW = 32
_P = 32768
_C = 2048
_NCHUNK = _P // _C
_ROWS = _C // 128
_MAGIC = 8388608.0


def _to_index(v, a, scale):
    xn = v / a
    xn = (xn + 1.0) / 2.0
    t = xn * 2.0 - 1.0
    w = (t + 1.0) * 0.5 * scale
    w = jnp.maximum(w, 0.0)
    w = jnp.minimum(w, scale)
    r = (w + _MAGIC) - _MAGIC
    return r.astype(jnp.int32)


def _body(x0_hbm, x1_hbm, x2_hbm, vol_hbm, aabb_hbm, out_hbm,
          xa0, xa1, xa2, xb0, xb1, xb2, abuf,
          idxa, idxb_, outa, outb_, sem_x, sem_g, sem_o):
    cid = lax.axis_index("c")
    sid = lax.axis_index("s")
    wid = sid * 2 + cid
    base = jnp.minimum(wid * _P, _N - _P)

    xbufs = ((xa0, xa1, xa2), (xb0, xb1, xb2))
    idxbs = (idxa, idxb_)
    outbs = (outa, outb_)

    pltpu.sync_copy(aabb_hbm, abuf)
    a0 = abuf[0, :]
    a1 = abuf[1, :]
    a2 = abuf[2, :]

    def fire_x(k, p):
        cbase = base + k * _C
        return [
            pltpu.async_copy(h.at[pl.ds(cbase, _C)], xbufs[p][c], sem_x)
            for c, h in enumerate((x0_hbm, x1_hbm, x2_hbm))
        ]

    def compute(p):
        vxb, vyb, vzb = xbufs[p]
        idx = idxbs[p]

        @plsc.parallel_loop(0, _C // 16, unroll=4)
        def _(i):
            off = i * 16
            vx = vxb[pl.ds(off, 16)]
            vy = vyb[pl.ds(off, 16)]
            vz = vzb[pl.ds(off, 16)]
            iz = _to_index(vx, a0, 127.0)
            iy = _to_index(vy, a1, 127.0)
            ix = _to_index(vz, a2, 95.0)
            flat = (iz * _H + iy) * _W + ix
            idx[i // 8, pl.ds((i % 8) * 16, 16)] = flat

    def fire_gathers(p):
        return [
            pltpu.async_copy(vol_hbm.at[idxbs[p].at[r]],
                             outbs[p].at[pl.ds(r * 128, 128)], sem_g)
            for r in range(_ROWS)
        ]

    def fire_out(k, p):
        cbase = base + k * _C
        return pltpu.async_copy(outbs[p], out_hbm.at[pl.ds(cbase, _C)],
                                sem_o)

    for c in fire_x(0, 0):
        c.wait()
    compute(0)

    pend_out = [None, None]
    for k in range(_NCHUNK):
        p = k % 2
        pn = (k + 1) % 2
        if pend_out[p] is not None:
            pend_out[p].wait()
            pend_out[p] = None
        gathers = fire_gathers(p)
        if k < _NCHUNK - 1:
            for c in fire_x(k + 1, pn):
                c.wait()
            compute(pn)
        for c in gathers:
            c.wait()
        pend_out[p] = fire_out(k, p)
    for p in range(2):
        if pend_out[p] is not None:
            pend_out[p].wait()


@jax.jit
def _sc_lookup(x0, x1, x2, vol, ab):
    mesh = plsc.VectorSubcoreMesh(core_axis_name="c", subcore_axis_name="s")
    run = functools.partial(
        pl.kernel,
        mesh=mesh,
        compiler_params=pltpu.CompilerParams(needs_layout_passes=False),
        out_type=jax.ShapeDtypeStruct((_N,), jnp.float32),
        scratch_types=[
            pltpu.VMEM((_C,), jnp.float32),
            pltpu.VMEM((_C,), jnp.float32),
            pltpu.VMEM((_C,), jnp.float32),
            pltpu.VMEM((_C,), jnp.float32),
            pltpu.VMEM((_C,), jnp.float32),
            pltpu.VMEM((_C,), jnp.float32),
            pltpu.VMEM((3, 16), jnp.float32),
            pltpu.VMEM((_ROWS, 128), jnp.int32),
            pltpu.VMEM((_ROWS, 128), jnp.int32),
            pltpu.VMEM((_C,), jnp.float32),
            pltpu.VMEM((_C,), jnp.float32),
            pltpu.SemaphoreType.DMA,
            pltpu.SemaphoreType.DMA,
            pltpu.SemaphoreType.DMA,
        ],
    )(_body)
    return run(x0, x1, x2, vol, ab)


def kernel(x, aabb, image, voxels):
    vol = image.reshape(-1)
    ab = jnp.broadcast_to(aabb[:, None], (3, 16))
    out = _sc_lookup(x[:, 0], x[:, 1], x[:, 2], vol, ab)
    return out[:, None]


# speedup vs baseline: 23.8525x; 1.7388x over previous
import functools

import jax
import jax.numpy as jnp
from jax import lax
from jax.experimental import pallas as pl
from jax.experimental.pallas import tpu as pltpu
from jax.experimental.pallas import tpu_sc as plsc

_D, _H, _W = 128, 128, 96
_N = 1_000_000
_NW = 32
_P = 32768
_C = 2048
_NCHUNK = _P // _C
_ROWS = _C // 128
_MAGIC = 8388608.0


def _to_index(v, a, scale):
    xn = v / a
    xn = (xn + 1.0) / 2.0
    t = xn * 2.0 - 1.0
    w = (t + 1.0) * 0.5 * scale
    w = jnp.maximum(w, 0.0)
    w = jnp.minimum(w, scale)
    r = (w + _MAGIC) - _MAGIC
    return r.astype(jnp.int32)


def _body(x0_hbm, x1_hbm, x2_hbm, vol_hbm, aabb_hbm, out_hbm,
          xa0, xa1, xa2, xb0, xb1, xb2, abuf,
          idxa, idxb_, outa, outb_, vol_sp, sem_x, sem_g, sem_o):
    cid = lax.axis_index("c")
    sid = lax.axis_index("s")
    wid = sid * 2 + cid
    base = jnp.minimum(wid * _P, _N - _P)

    vshard = (_D * _H * _W) // 16
    pltpu.sync_copy(vol_hbm.at[pl.ds(sid * vshard, vshard)],
                    vol_sp.at[pl.ds(sid * vshard, vshard)])
    plsc.subcore_barrier()

    xbufs = ((xa0, xa1, xa2), (xb0, xb1, xb2))
    idxbs = (idxa, idxb_)
    outbs = (outa, outb_)

    pltpu.sync_copy(aabb_hbm, abuf)
    a0 = abuf[0, :]
    a1 = abuf[1, :]
    a2 = abuf[2, :]

    def fire_x(k, p):
        cbase = base + k * _C
        return [
            pltpu.async_copy(h.at[pl.ds(cbase, _C)], xbufs[p][c], sem_x)
            for c, h in enumerate((x0_hbm, x1_hbm, x2_hbm))
        ]

    def compute(p):
        vxb, vyb, vzb = xbufs[p]
        idx = idxbs[p]

        @plsc.parallel_loop(0, _C // 16, unroll=4)
        def _(i):
            off = i * 16
            vx = vxb[pl.ds(off, 16)]
            vy = vyb[pl.ds(off, 16)]
            vz = vzb[pl.ds(off, 16)]
            iz = _to_index(vx, a0, 127.0)
            iy = _to_index(vy, a1, 127.0)
            ix = _to_index(vz, a2, 95.0)
            flat = (iz * _H + iy) * _W + ix
            idx[i // 8, pl.ds((i % 8) * 16, 16)] = flat

    def fire_gathers(p):
        return [
            pltpu.async_copy(vol_sp.at[idxbs[p].at[r]],
                             outbs[p].at[pl.ds(r * 128, 128)], sem_g)
            for r in range(_ROWS)
        ]

    def fire_out(k, p):
        cbase = base + k * _C
        return pltpu.async_copy(outbs[p], out_hbm.at[pl.ds(cbase, _C)],
                                sem_o)

    for c in fire_x(0, 0):
        c.wait()
    compute(0)

    pend_out = [None, None]
    for k in range(_NCHUNK):
        p = k % 2
        pn = (k + 1) % 2
        if pend_out[p] is not None:
            pend_out[p].wait()
            pend_out[p] = None
        gathers = fire_gathers(p)
        if k < _NCHUNK - 1:
            for c in fire_x(k + 1, pn):
                c.wait()
            compute(pn)
        for c in gathers:
            c.wait()
        pend_out[p] = fire_out(k, p)
    for p in range(2):
        if pend_out[p] is not None:
            pend_out[p].wait()


@jax.jit
def _sc_lookup(x0, x1, x2, vol, ab):
    mesh = plsc.VectorSubcoreMesh(core_axis_name="c", subcore_axis_name="s")
    run = functools.partial(
        pl.kernel,
        mesh=mesh,
        compiler_params=pltpu.CompilerParams(needs_layout_passes=False),
        out_type=jax.ShapeDtypeStruct((_N,), jnp.float32),
        scratch_types=[
            pltpu.VMEM((_C,), jnp.float32),
            pltpu.VMEM((_C,), jnp.float32),
            pltpu.VMEM((_C,), jnp.float32),
            pltpu.VMEM((_C,), jnp.float32),
            pltpu.VMEM((_C,), jnp.float32),
            pltpu.VMEM((_C,), jnp.float32),
            pltpu.VMEM((3, 16), jnp.float32),
            pltpu.VMEM((_ROWS, 128), jnp.int32),
            pltpu.VMEM((_ROWS, 128), jnp.int32),
            pltpu.VMEM((_C,), jnp.float32),
            pltpu.VMEM((_C,), jnp.float32),
            pltpu.VMEM_SHARED((_D * _H * _W,), jnp.float32),
            pltpu.SemaphoreType.DMA,
            pltpu.SemaphoreType.DMA,
            pltpu.SemaphoreType.DMA,
        ],
    )(_body)
    return run(x0, x1, x2, vol, ab)


def kernel(x, aabb, image, voxels):
    vol = image.reshape(-1)
    ab = jnp.broadcast_to(aabb[:, None], (3, 16))
    out = _sc_lookup(x[:, 0], x[:, 1], x[:, 2], vol, ab)
    return out[:, None]


# speedup vs baseline: 31.7762x; 1.3322x over previous
import functools

import jax
import jax.numpy as jnp
from jax import lax
from jax.experimental import pallas as pl
from jax.experimental.pallas import tpu as pltpu
from jax.experimental.pallas import tpu_sc as plsc

_D, _H, _W = 128, 128, 96
_N = 1_000_000
_NW = 32
_P = 32768
_C = 2048
_NCHUNK = _P // _C
_ROWS = _C // 128
_TAIL_BASE = 967296
_TAIL_LEN = _N - (_TAIL_BASE + (_NCHUNK - 1) * _C)
_MAGIC = 8388608.0


def _to_index(v, a, scale):
    xn = v / a
    xn = (xn + 1.0) / 2.0
    t = xn * 2.0 - 1.0
    w = (t + 1.0) * 0.5 * scale
    w = jnp.maximum(w, 0.0)
    w = jnp.minimum(w, scale)
    r = (w + _MAGIC) - _MAGIC
    return r.astype(jnp.int32)


def _body(xt_hbm, vol_hbm, aabb_hbm, out_hbm,
          xta, xtb, abuf,
          idxa, idxb_, outa, outb_, vol_sp, sem_x, sem_g, sem_o):
    cid = lax.axis_index("c")
    sid = lax.axis_index("s")
    wid = sid * 2 + cid
    base = jnp.minimum(wid * _P, _TAIL_BASE)
    is_tail = wid * _P > _TAIL_BASE

    vshard = (_D * _H * _W) // 16
    pltpu.sync_copy(vol_hbm.at[pl.ds(sid * vshard, vshard)],
                    vol_sp.at[pl.ds(sid * vshard, vshard)])
    plsc.subcore_barrier()

    xbufs = (xta, xtb)
    idxbs = (idxa, idxb_)
    outbs = (outa, outb_)

    pltpu.sync_copy(aabb_hbm, abuf)
    a0 = abuf[0, :]
    a1 = abuf[1, :]
    a2 = abuf[2, :]

    def fire_x(k, p):
        cbase = pl.multiple_of(base + k * _C, 128)
        return [
            pltpu.async_copy(xt_hbm.at[:, pl.ds(cbase, _C)], xbufs[p],
                             sem_x)
        ]

    def compute(p):
        xb = xbufs[p]
        idx = idxbs[p]

        @plsc.parallel_loop(0, _C // 16, unroll=4)
        def _(i):
            off = i * 16
            vx = xb[0, pl.ds(off, 16)]
            vy = xb[1, pl.ds(off, 16)]
            vz = xb[2, pl.ds(off, 16)]
            iz = _to_index(vx, a0, 127.0)
            iy = _to_index(vy, a1, 127.0)
            ix = _to_index(vz, a2, 95.0)
            flat = (iz * _H + iy) * _W + ix
            flat = jnp.minimum(jnp.maximum(flat, 0), _D * _H * _W - 1)
            idx[i // 8, pl.ds((i % 8) * 16, 16)] = flat

    def fire_gathers(p):
        return [
            pltpu.async_copy(vol_sp.at[idxbs[p].at[r]],
                             outbs[p].at[pl.ds(r * 128, 128)], sem_g)
            for r in range(_ROWS)
        ]

    def fire_out(k, p):
        cbase = base + k * _C
        return pltpu.async_copy(outbs[p], out_hbm.at[pl.ds(cbase, _C)],
                                sem_o)

    for c in fire_x(0, 0):
        c.wait()
    compute(0)

    pend_out = [None, None]
    for k in range(_NCHUNK):
        p = k % 2
        pn = (k + 1) % 2
        if pend_out[p] is not None:
            pend_out[p].wait()
            pend_out[p] = None
        gathers = fire_gathers(p)
        if k < _NCHUNK - 1:
            for c in fire_x(k + 1, pn):
                c.wait()
            compute(pn)
        for c in gathers:
            c.wait()
        if k < _NCHUNK - 1:
            pend_out[p] = fire_out(k, p)
        else:
            cbase = base + k * _C

            @pl.when(is_tail)
            def _():
                pltpu.sync_copy(outbs[p].at[pl.ds(0, _TAIL_LEN)],
                                out_hbm.at[pl.ds(cbase, _TAIL_LEN)])

            @pl.when(jnp.logical_not(is_tail))
            def _():
                pltpu.sync_copy(outbs[p], out_hbm.at[pl.ds(cbase, _C)])
    for p in range(2):
        if pend_out[p] is not None:
            pend_out[p].wait()


@jax.jit
def _sc_lookup(xt, vol, ab):
    mesh = plsc.VectorSubcoreMesh(core_axis_name="c", subcore_axis_name="s")
    run = functools.partial(
        pl.kernel,
        mesh=mesh,
        compiler_params=pltpu.CompilerParams(needs_layout_passes=False),
        out_type=jax.ShapeDtypeStruct((_N,), jnp.float32),
        scratch_types=[
            pltpu.VMEM((3, _C), jnp.float32),
            pltpu.VMEM((3, _C), jnp.float32),
            pltpu.VMEM((3, 16), jnp.float32),
            pltpu.VMEM((_ROWS, 128), jnp.int32),
            pltpu.VMEM((_ROWS, 128), jnp.int32),
            pltpu.VMEM((_C,), jnp.float32),
            pltpu.VMEM((_C,), jnp.float32),
            pltpu.VMEM_SHARED((_D * _H * _W,), jnp.float32),
            pltpu.SemaphoreType.DMA,
            pltpu.SemaphoreType.DMA,
            pltpu.SemaphoreType.DMA,
        ],
    )(_body)
    return run(xt, vol, ab)


def kernel(x, aabb, image, voxels):
    vol = image.reshape(-1)
    ab = jnp.broadcast_to(aabb[:, None], (3, 16))
    out = _sc_lookup(x.T, vol, ab)
    return out[:, None]
